# Initial kernel scaffold; baseline (speedup 1.0000x reference)
import jax
import jax.numpy as jnp
from jax.experimental import pallas as pl


def kernel(input_ids, fix_offsets, fix_words, table, word_embeddings):
    raise NotImplementedError("write your pallas kernel here")


# speedup vs baseline: 2.1685x; 2.1685x over previous
import functools

import jax
import jax.numpy as jnp
from jax import lax
from jax.experimental import pallas as pl
from jax.experimental.pallas import tpu as pltpu
from jax.experimental.pallas import tpu_sc as plsc

NC, NS, LANES = 2, 16, 16
NW = NC * NS

B, L, V, D = 4096, 200, 1000000, 64
F = 4
CHUNK = 128
TOK_PER_W = (B * L) // NW
NCHUNK = TOK_PER_W // CHUNK
FIX_ROWS = (B // NW) * F // CHUNK


def _sc_body(ids_hbm, table_hbm, tgt_hbm, words_hbm, we_hbm, out_hbm,
             idx_v, rows_v, words_v, tgt_v, fvecs_v, gsem, ssem):
    c = lax.axis_index("c")
    s = lax.axis_index("s")
    w = s * NC + c

    pltpu.sync_copy(ids_hbm.at[w], idx_v)

    @pl.loop(0, NCHUNK)
    def _gather_chunk(g):
        pltpu.async_copy(table_hbm.at[idx_v.at[g]], rows_v, gsem).wait()
        pltpu.sync_copy(rows_v, out_hbm.at[pl.ds((w * NCHUNK + g) * CHUNK, CHUNK)])

    pltpu.sync_copy(words_hbm.at[w], words_v)
    pltpu.sync_copy(tgt_hbm.at[w], tgt_v)
    for j in range(FIX_ROWS):
        pltpu.async_copy(we_hbm.at[words_v.at[j]], fvecs_v, gsem).wait()
        pltpu.async_copy(fvecs_v, out_hbm.at[tgt_v.at[j]], ssem).wait()


@jax.jit
def _embed_with_fixes(ids3, table, tgt3, words3, word_embeddings):
    mesh = plsc.VectorSubcoreMesh(
        core_axis_name="c", subcore_axis_name="s",
        num_cores=NC, num_subcores=NS)
    return pl.kernel(
        _sc_body,
        out_type=jax.ShapeDtypeStruct((B * L, D), jnp.float32),
        mesh=mesh,
        compiler_params=pltpu.CompilerParams(use_tc_tiling_on_sc=False),
        scratch_types=[
            pltpu.VMEM((NCHUNK, CHUNK), jnp.int32),
            pltpu.VMEM((CHUNK, D), jnp.float32),
            pltpu.VMEM((FIX_ROWS, CHUNK), jnp.int32),
            pltpu.VMEM((FIX_ROWS, CHUNK), jnp.int32),
            pltpu.VMEM((CHUNK, D), jnp.float32),
            pltpu.SemaphoreType.DMA,
            pltpu.SemaphoreType.DMA,
        ],
    )(ids3, table, tgt3, words3, word_embeddings)


def kernel(input_ids, fix_offsets, fix_words, table, word_embeddings):
    ids3 = input_ids.reshape(NW, NCHUNK, CHUNK)

    f_ids = jnp.arange(F, dtype=jnp.int32)
    eq = fix_offsets[:, :, None] == fix_offsets[:, None, :]
    last = jnp.max(jnp.where(eq, f_ids[None, None, :], -1), axis=2)
    win_words = jnp.take_along_axis(fix_words, last, axis=1)

    tgt = jnp.arange(B, dtype=jnp.int32)[:, None] * L + fix_offsets
    tgt3 = tgt.reshape(NW, FIX_ROWS, CHUNK)
    words3 = win_words.reshape(NW, FIX_ROWS, CHUNK)

    out = _embed_with_fixes(ids3, table, tgt3, words3, word_embeddings)
    return out.reshape(B, L, D)


# speedup vs baseline: 2.4116x; 1.1121x over previous
import functools

import jax
import jax.numpy as jnp
from jax import lax
from jax.experimental import pallas as pl
from jax.experimental.pallas import tpu as pltpu
from jax.experimental.pallas import tpu_sc as plsc

NC, NS, LANES = 2, 16, 16
NW = NC * NS

B, L, V, D = 4096, 200, 1000000, 64
F = 4
CHUNK = 128
TOK_PER_W = (B * L) // NW
NCHUNK = TOK_PER_W // CHUNK
FIX_ROWS = (B // NW) * F // CHUNK


NBUF = 8


def _sc_body(ids_hbm, table_hbm, tgt_hbm, words_hbm, we_hbm, out_hbm,
             idx_v, rows_v, words_v, tgt_v, fvecs_v, gsem, ssem):
    c = lax.axis_index("c")
    s = lax.axis_index("s")
    w = s * NC + c

    pltpu.sync_copy(ids_hbm.at[w], idx_v)

    def out_slice(chunk):
        return out_hbm.at[pl.ds((w * NCHUNK + chunk) * CHUNK, CHUNK)]

    def fire_gather(chunk, b):
        pltpu.async_copy(table_hbm.at[idx_v.at[chunk]], rows_v.at[b],
                         gsem.at[b])

    def wait_gather(b):
        pltpu.make_async_copy(table_hbm.at[idx_v.at[0]], rows_v.at[b],
                              gsem.at[b]).wait()

    def fire_store(chunk, b):
        pltpu.async_copy(rows_v.at[b], out_slice(chunk), ssem.at[b])

    def wait_store(b):
        pltpu.make_async_copy(rows_v.at[b], out_slice(0), ssem.at[b]).wait()

    for b in range(NBUF):
        fire_gather(b, b)

    @pl.loop(0, NCHUNK, step=NBUF)
    def _group(g0):
        for b in range(NBUF):
            chunk = g0 + b
            wait_gather(b)
            fire_store(chunk, b)

            @pl.when(chunk + NBUF < NCHUNK)
            def _refill():
                wait_store(b)
                fire_gather(chunk + NBUF, b)

    for b in range(NBUF):
        wait_store(b)

    pltpu.sync_copy(words_hbm.at[w], words_v)
    pltpu.sync_copy(tgt_hbm.at[w], tgt_v)
    for j in range(FIX_ROWS):
        pltpu.async_copy(we_hbm.at[words_v.at[j]], fvecs_v, gsem.at[0]).wait()
        pltpu.async_copy(fvecs_v, out_hbm.at[tgt_v.at[j]], ssem.at[0]).wait()


@jax.jit
def _embed_with_fixes(ids3, table, tgt3, words3, word_embeddings):
    mesh = plsc.VectorSubcoreMesh(
        core_axis_name="c", subcore_axis_name="s",
        num_cores=NC, num_subcores=NS)
    return pl.kernel(
        _sc_body,
        out_type=jax.ShapeDtypeStruct((B * L, D), jnp.float32),
        mesh=mesh,
        compiler_params=pltpu.CompilerParams(use_tc_tiling_on_sc=False),
        scratch_types=[
            pltpu.VMEM((NCHUNK, CHUNK), jnp.int32),
            pltpu.VMEM((NBUF, CHUNK, D), jnp.float32),
            pltpu.VMEM((FIX_ROWS, CHUNK), jnp.int32),
            pltpu.VMEM((FIX_ROWS, CHUNK), jnp.int32),
            pltpu.VMEM((CHUNK, D), jnp.float32),
            pltpu.SemaphoreType.DMA((NBUF,)),
            pltpu.SemaphoreType.DMA((NBUF,)),
        ],
    )(ids3, table, tgt3, words3, word_embeddings)


def kernel(input_ids, fix_offsets, fix_words, table, word_embeddings):
    ids3 = input_ids.reshape(NW, NCHUNK, CHUNK)

    f_ids = jnp.arange(F, dtype=jnp.int32)
    eq = fix_offsets[:, :, None] == fix_offsets[:, None, :]
    last = jnp.max(jnp.where(eq, f_ids[None, None, :], -1), axis=2)
    win_words = jnp.take_along_axis(fix_words, last, axis=1)

    tgt = jnp.arange(B, dtype=jnp.int32)[:, None] * L + fix_offsets
    tgt3 = tgt.reshape(NW, FIX_ROWS, CHUNK)
    words3 = win_words.reshape(NW, FIX_ROWS, CHUNK)

    out = _embed_with_fixes(ids3, table, tgt3, words3, word_embeddings)
    return out.reshape(B, L, D)
